# baseline (speedup 1.0000x reference)
import jax
import jax.numpy as jnp
from jax import lax
from jax.experimental import pallas as pl
from jax.experimental.pallas import tpu as pltpu
from jax.experimental.pallas import tpu_sc as plsc

_S = 768
_GL = 3
_NT = 16
_CW = 48
_DV = 8
_NP = 9


def _sc_body(a_hbm, out_hbm, strip0_v, strip1_v, outb0_v, outb1_v, sem_in, sem_out):
    wid = lax.axis_index("c") * 16 + lax.axis_index("s")
    batch = wid // 16
    h = wid - batch * 16
    th = h // 8
    hh = h - th * 8

    lane = lax.iota(jnp.int32, 16)
    r3 = lane * 3
    strips = (strip0_v, strip1_v)
    outbs = (outb0_v, outb1_v)

    def src(t):
        return a_hbm.at[wid, :, pl.ds(t * _CW, _CW)]

    def dst(t, p):
        return out_hbm.at[
            batch, p, th, pl.ds(t * 2 * _DV, 2 * _DV), pl.ds(hh, 1), :
        ]

    pltpu.async_copy(src(0), strip0_v, sem_in)

    def strip_pair_loop(t2, carry):
        for par in range(2):
            t = 2 * t2 + par
            cur = strips[par]

            @pl.when(t + 1 < _NT)
            def _prefetch():
                pltpu.async_copy(src(t + 1), strips[1 - par], sem_in)

            pltpu.make_async_copy(src(t), cur, sem_in).wait()

            for h2 in range(2):
                q = 2 * t + h2
                outb = outbs[h2]

                @pl.when(t >= 1)
                def _drain_prev():
                    for p in range(_NP):
                        pltpu.make_async_copy(
                            outb.at[p], dst(q - 2, p), sem_out
                        ).wait()

                def lane_loop(s, inner):
                    ti_off = s >> 3
                    c_off = (s & 7) * 16
                    rbase = r3 + s * 48
                    for bp in range(_GL):
                        rvec = rbase + bp
                        for l in range(_GL):
                            p = bp * _GL + l
                            for v in range(_DV):
                                cvec = jnp.full(
                                    (16,), 24 * h2 + _GL * v + l, jnp.int32
                                )
                                val = plsc.load_gather(cur, [rvec, cvec])
                                outb[p, 2 * v + ti_off, 0, pl.ds(c_off, 16)] = val
                    return inner

                lax.fori_loop(0, 16, lane_loop, 0)
                for p in range(_NP):
                    pltpu.async_copy(outb.at[p], dst(q, p), sem_out)
        return carry

    lax.fori_loop(0, _NT // 2, strip_pair_loop, 0)
    for qq in (2 * _NT - 2, 2 * _NT - 1):
        for p in range(_NP):
            pltpu.make_async_copy(outbs[qq % 2].at[p], dst(qq, p), sem_out).wait()


def kernel(inputs):
    A = inputs
    B, H, S1, S2 = A.shape
    a = A.reshape(B * H, S1, S2)
    mesh = plsc.VectorSubcoreMesh(
        core_axis_name="c", subcore_axis_name="s", num_cores=2, num_subcores=16
    )
    f = pl.kernel(
        _sc_body,
        mesh=mesh,
        compiler_params=pltpu.CompilerParams(
            use_tc_tiling_on_sc=False, needs_layout_passes=False
        ),
        out_type=jax.ShapeDtypeStruct((B, _NP, 2, 512, 8, 128), jnp.float32),
        scratch_types=[
            pltpu.VMEM((_S, _CW), jnp.float32),
            pltpu.VMEM((_S, _CW), jnp.float32),
            pltpu.VMEM((_NP, 2 * _DV, 1, 128), jnp.float32),
            pltpu.VMEM((_NP, 2 * _DV, 1, 128), jnp.float32),
            pltpu.SemaphoreType.DMA,
            pltpu.SemaphoreType.DMA,
        ],
    )
    out = f(a)
    o = out.transpose(0, 1, 2, 4, 3, 5).reshape(B, _GL, _GL, H, 65536)
    return o.transpose(0, 3, 4, 1, 2)


# speedup vs baseline: 2.0673x; 2.0673x over previous
import jax
import jax.numpy as jnp
from jax import lax
from jax.experimental import pallas as pl
from jax.experimental.pallas import tpu as pltpu
from jax.experimental.pallas import tpu_sc as plsc

_S = 768
_GL = 3
_NT = 32
_CW = 24
_DV = 8
_NP = 9


def _sc_body(a_hbm, out_hbm, strip0_v, strip1_v, outb0_v, outb1_v, sem_in, sem_out):
    wid = lax.axis_index("c") * 16 + lax.axis_index("s")
    batch = wid // 16
    h = wid - batch * 16
    th = h // 8
    hh = h - th * 8

    lane = lax.iota(jnp.int32, 16)
    r3 = lane * 3
    strips = (strip0_v, strip1_v)
    outbs = (outb0_v, outb1_v)

    def src(t):
        return a_hbm.at[wid, :, pl.ds(t * _CW, _CW)]

    def dst(t, p):
        return out_hbm.at[
            batch, p, th, pl.ds(t * 2 * _DV, 2 * _DV), pl.ds(hh, 1), :
        ]

    pltpu.async_copy(src(0), strip0_v, sem_in)

    def strip_pair_loop(t2, carry):
        for par in range(2):
            t = 2 * t2 + par
            cur = strips[par]

            @pl.when(t + 1 < _NT)
            def _prefetch():
                pltpu.async_copy(src(t + 1), strips[1 - par], sem_in)

            pltpu.make_async_copy(src(t), cur, sem_in).wait()

            outb = outbs[par]

            @pl.when(t >= 2)
            def _drain_prev():
                for p in range(_NP):
                    pltpu.make_async_copy(outb.at[p], dst(t - 2, p), sem_out).wait()

            @plsc.parallel_loop(0, 16, step=1)
            def lane_loop(s):
                ti_off = s >> 3
                c_off = (s & 7) * 16
                rbase = r3 + s * 48
                for bp in range(_GL):
                    rvec = rbase + bp
                    for l in range(_GL):
                        p = bp * _GL + l
                        for v in range(_DV):
                            cvec = jnp.full((16,), _GL * v + l, jnp.int32)
                            val = plsc.load_gather(cur, [rvec, cvec])
                            outb[p, 2 * v + ti_off, 0, pl.ds(c_off, 16)] = val

            for p in range(_NP):
                pltpu.async_copy(outb.at[p], dst(t, p), sem_out)
        return carry

    lax.fori_loop(0, _NT // 2, strip_pair_loop, 0)
    for tt in (_NT - 2, _NT - 1):
        for p in range(_NP):
            pltpu.make_async_copy(outbs[tt % 2].at[p], dst(tt, p), sem_out).wait()


def kernel(inputs):
    A = inputs
    B, H, S1, S2 = A.shape
    a = A.reshape(B * H, S1, S2)
    mesh = plsc.VectorSubcoreMesh(
        core_axis_name="c", subcore_axis_name="s", num_cores=2, num_subcores=16
    )
    f = pl.kernel(
        _sc_body,
        mesh=mesh,
        compiler_params=pltpu.CompilerParams(
            use_tc_tiling_on_sc=False, needs_layout_passes=False
        ),
        out_type=jax.ShapeDtypeStruct((B, _NP, 2, 512, 8, 128), jnp.float32),
        scratch_types=[
            pltpu.VMEM((_S, _CW), jnp.float32),
            pltpu.VMEM((_S, _CW), jnp.float32),
            pltpu.VMEM((_NP, 2 * _DV, 1, 128), jnp.float32),
            pltpu.VMEM((_NP, 2 * _DV, 1, 128), jnp.float32),
            pltpu.SemaphoreType.DMA,
            pltpu.SemaphoreType.DMA,
        ],
    )
    out = f(a)
    o = out.transpose(0, 1, 2, 4, 3, 5).reshape(B, _GL, _GL, H, 65536)
    return o.transpose(0, 3, 4, 1, 2)


# speedup vs baseline: 2.8221x; 1.3651x over previous
import jax
import jax.numpy as jnp
from jax import lax
from jax.experimental import pallas as pl
from jax.experimental.pallas import tpu as pltpu
from jax.experimental.pallas import tpu_sc as plsc

_S = 768
_GL = 3
_NT = 32
_CW = 24
_DV = 8
_NP = 9


def _sc_body(a_hbm, out_hbm, strip0_v, strip1_v, outb0_v, outb1_v, sem_in, sem_out):
    wid = lax.axis_index("c") * 16 + lax.axis_index("s")
    batch = wid // 16
    h = wid - batch * 16
    th = h // 8
    hh = h - th * 8

    lane = lax.iota(jnp.int32, 16)
    r3 = lane * 3
    strips = (strip0_v, strip1_v)
    outbs = (outb0_v, outb1_v)

    def fetch(t, buf):
        ct = (t * _CW) // 128
        cc = (t * _CW) - ct * 128

        @pl.when(cc <= 104)
        def _one():
            pltpu.async_copy(a_hbm.at[wid, :, ct, :, pl.ds(cc, 24)], buf, sem_in)

        @pl.when(cc == 112)
        def _split16():
            pltpu.async_copy(
                a_hbm.at[wid, :, ct, :, pl.ds(112, 16)],
                buf.at[:, :, pl.ds(0, 16)], sem_in,
            )
            pltpu.async_copy(
                a_hbm.at[wid, :, ct + 1, :, pl.ds(0, 8)],
                buf.at[:, :, pl.ds(16, 8)], sem_in,
            )

        @pl.when(cc == 120)
        def _split8():
            pltpu.async_copy(
                a_hbm.at[wid, :, ct, :, pl.ds(120, 8)],
                buf.at[:, :, pl.ds(0, 8)], sem_in,
            )
            pltpu.async_copy(
                a_hbm.at[wid, :, ct + 1, :, pl.ds(0, 16)],
                buf.at[:, :, pl.ds(8, 16)], sem_in,
            )

    def wait_fetch(buf):
        pltpu.make_async_copy(
            a_hbm.at[wid, :, 0, :, pl.ds(0, 24)], buf, sem_in
        ).wait()

    def dst(t, p):
        return out_hbm.at[
            batch, p, th, pl.ds(t * 2 * _DV, 2 * _DV), pl.ds(hh, 1), :
        ]

    fetch(0, strip0_v)

    def strip_pair_loop(t2, carry):
        for par in range(2):
            t = 2 * t2 + par
            cur = strips[par]

            @pl.when(t + 1 < _NT)
            def _prefetch():
                fetch(t + 1, strips[1 - par])

            wait_fetch(cur)

            outb = outbs[par]

            @pl.when(t >= 2)
            def _drain_prev():
                for p in range(_NP):
                    pltpu.make_async_copy(outb.at[p], dst(t - 2, p), sem_out).wait()

            @plsc.parallel_loop(0, 16, step=1)
            def lane_loop(s):
                ti_off = s >> 3
                c_off = (s & 7) * 16
                rbase = r3 + s * 48
                for bp in range(_GL):
                    rvec = rbase + bp
                    tr = rvec >> 3
                    rr = rvec & 7
                    for l in range(_GL):
                        p = bp * _GL + l
                        for v in range(_DV):
                            cvec = jnp.full((16,), _GL * v + l, jnp.int32)
                            val = plsc.load_gather(cur, [tr, rr, cvec])
                            outb[p, 2 * v + ti_off, 0, pl.ds(c_off, 16)] = val

            for p in range(_NP):
                pltpu.async_copy(outb.at[p], dst(t, p), sem_out)
        return carry

    lax.fori_loop(0, _NT // 2, strip_pair_loop, 0)
    for tt in (_NT - 2, _NT - 1):
        for p in range(_NP):
            pltpu.make_async_copy(outbs[tt % 2].at[p], dst(tt, p), sem_out).wait()


def kernel(inputs):
    A = inputs
    B, H, S1, S2 = A.shape
    a = (
        A.reshape(B * H, S1 // 8, 8, S2 // 128, 128)
        .transpose(0, 1, 3, 2, 4)
        .reshape(B * H, S1 // 8, S2 // 128, 8, 128)
    )
    mesh = plsc.VectorSubcoreMesh(
        core_axis_name="c", subcore_axis_name="s", num_cores=2, num_subcores=16
    )
    f = pl.kernel(
        _sc_body,
        mesh=mesh,
        compiler_params=pltpu.CompilerParams(
            use_tc_tiling_on_sc=False, needs_layout_passes=False
        ),
        out_type=jax.ShapeDtypeStruct((B, _NP, 2, 512, 8, 128), jnp.float32),
        scratch_types=[
            pltpu.VMEM((_S // 8, 8, _CW), jnp.float32),
            pltpu.VMEM((_S // 8, 8, _CW), jnp.float32),
            pltpu.VMEM((_NP, 2 * _DV, 1, 128), jnp.float32),
            pltpu.VMEM((_NP, 2 * _DV, 1, 128), jnp.float32),
            pltpu.SemaphoreType.DMA,
            pltpu.SemaphoreType.DMA,
        ],
    )
    out = f(a)
    o = out.transpose(0, 1, 2, 4, 3, 5).reshape(B, _GL, _GL, H, 65536)
    return o.transpose(0, 3, 4, 1, 2)
